# baseline (speedup 1.0000x reference)
import jax
import jax.numpy as jnp
from jax.experimental import pallas as pl

K = 32


def _gnn_body(x_ref, o_ref):
    x = x_ref[...]
    lanes = x.shape[-1]
    mask = (jax.lax.iota(jnp.int32, lanes) < K)
    weighted = jnp.where(mask[None, :], x, 0.0) * (1.0 / K)
    m = jnp.sum(weighted, axis=-1)
    o_ref[...] = jnp.broadcast_to(m[:, None], o_ref.shape)


def kernel(feat_pos):
    b, c, h, w = feat_pos.shape
    n = h * w
    flat = feat_pos.reshape(b * c, n)
    out = pl.pallas_call(
        _gnn_body,
        grid=(1,),
        in_specs=[pl.BlockSpec((b * c, 128), lambda i: (0, 0))],
        out_specs=pl.BlockSpec((b * c, n), lambda i: (0, 0)),
        out_shape=jax.ShapeDtypeStruct((b * c, n), feat_pos.dtype),
    )(flat)
    return out.reshape(b, c, h, w)


# speedup vs baseline: 3.0429x; 3.0429x over previous
import jax
import jax.numpy as jnp
from jax.experimental import pallas as pl

K = 32


def _gnn_body(x_ref, o_ref):
    x = x_ref[...]
    lanes = x.shape[-1]
    mask = (jax.lax.iota(jnp.int32, lanes) < K)
    weighted = jnp.where(mask[None, None, :], x, 0.0) * (1.0 / K)
    m = jnp.sum(weighted, axis=-1)
    o_ref[...] = jnp.broadcast_to(m[..., None], o_ref.shape)


def kernel(feat_pos):
    b, c, h, w = feat_pos.shape
    n = h * w
    flat = feat_pos.reshape(b, c, n)
    out = pl.pallas_call(
        _gnn_body,
        grid=(1,),
        in_specs=[pl.BlockSpec((b, c, 128), lambda i: (0, 0, 0))],
        out_specs=pl.BlockSpec((b, c, n), lambda i: (0, 0, 0)),
        out_shape=jax.ShapeDtypeStruct((b, c, n), feat_pos.dtype),
    )(flat)
    return out.reshape(b, c, h, w)


# speedup vs baseline: 3.0758x; 1.0108x over previous
import jax
import jax.numpy as jnp
from jax.experimental import pallas as pl

K = 32


def _gnn_body(x_ref, o_ref):
    x = x_ref[...]
    lanes = x.shape[-1]
    mask = (jax.lax.iota(jnp.int32, lanes) < K)
    weighted = jnp.where(mask[None, None, :], x, 0.0) * (1.0 / K)
    m = jnp.sum(weighted, axis=-1)
    o_ref[...] = jnp.broadcast_to(m[..., None], o_ref.shape)


def kernel(feat_pos):
    b, c, h, w = feat_pos.shape
    n = h * w
    flat = feat_pos.reshape(b, c, n)
    out = pl.pallas_call(
        _gnn_body,
        grid=(2,),
        in_specs=[pl.BlockSpec((b, c, 128), lambda i: (0, 0, 0))],
        out_specs=pl.BlockSpec((b, c, n // 2), lambda i: (0, 0, i)),
        out_shape=jax.ShapeDtypeStruct((b, c, n), feat_pos.dtype),
    )(flat)
    return out.reshape(b, c, h, w)
